# baseline (speedup 1.0000x reference)
import functools

import jax
import jax.numpy as jnp
from jax import lax
from jax.experimental import pallas as pl
from jax.experimental.pallas import tpu as pltpu
from jax.experimental.pallas import tpu_sc as plsc

N = 10000
E = 320000
D = 128
R = 16
DOUT = 64

NC = 2
NS = 16
CH = 128
CPT = 80
CPT_A = 120
CPT_B = 40
CPT_MAX = max(CPT_A, CPT_B, 1)
EP = NC * NS * CPT * CH
NROWS = EP // CH
NH = 10112
PAD_ROW = N
NHIST = 10240
EPT = EP // (NC * NS)


def _agg_body(msg_hbm, src_hbm, dst_hbm, zeros_hbm, out_hbm,
              src_v, dst_v, b0, shared, s0):
    c = lax.axis_index("c")
    s = lax.axis_index("s")
    rows_per_tile = NH // NS
    r0 = s * rows_per_tile
    pltpu.sync_copy(zeros_hbm, shared.at[pl.ds(r0, rows_per_tile)])

    def run(t0, cpt):
        pltpu.sync_copy(src_hbm.at[pl.ds(t0, cpt)], src_v.at[pl.ds(0, cpt)])
        pltpu.sync_copy(dst_hbm.at[pl.ds(t0, cpt)], dst_v.at[pl.ds(0, cpt)])
        plsc.subcore_barrier()

        def body(j, carry):
            pltpu.async_copy(msg_hbm.at[src_v.at[j]], b0, s0).wait()
            pltpu.sync_copy(b0, shared.at[dst_v.at[j]], add=True)
            return carry

        lax.fori_loop(0, cpt, body, 0)

    @pl.when(c == 0)
    def _():
        run(s * CPT_A, CPT_A)

    if CPT_B:
        @pl.when(c == 1)
        def _():
            run(NS * CPT_A + s * CPT_B, CPT_B)

    plsc.subcore_barrier()
    pltpu.sync_copy(shared.at[pl.ds(r0, rows_per_tile)],
                    out_hbm.at[c, pl.ds(r0, rows_per_tile)])


def _deg_body(src_hbm, dst_hbm, out_hbm, src_v, dst_v, hist_v):
    c = lax.axis_index("c")
    s = lax.axis_index("s")
    wid = s * NC + c
    pltpu.sync_copy(src_hbm.at[wid], src_v)
    pltpu.sync_copy(dst_hbm.at[wid], dst_v)

    zeros16 = jnp.zeros((16,), jnp.float32)
    ones16 = jnp.ones((16,), jnp.float32)

    def zbody(g, carry):
        hist_v[pl.ds(g * 16, 16)] = zeros16
        return carry

    lax.fori_loop(0, (2 * NHIST) // 16, zbody, 0)

    def body(g, carry):
        si = src_v[pl.ds(g * 16, 16)]
        di = dst_v[pl.ds(g * 16, 16)] + NHIST
        plsc.addupdate_scatter(hist_v, [si], ones16)
        plsc.addupdate_scatter(hist_v, [di], ones16)
        return carry

    lax.fori_loop(0, EPT // 16, body, 0)
    pltpu.sync_copy(hist_v, out_hbm.at[wid])


@functools.lru_cache(maxsize=None)
def _sc_kernels():
    mesh = plsc.VectorSubcoreMesh(core_axis_name="c", subcore_axis_name="s")
    deg = pl.kernel(
        _deg_body,
        mesh=mesh,
        out_type=jax.ShapeDtypeStruct((NC * NS, 2 * NHIST), jnp.float32),
        scratch_types=[
            pltpu.VMEM((EPT,), jnp.int32),
            pltpu.VMEM((EPT,), jnp.int32),
            pltpu.VMEM((2 * NHIST,), jnp.float32),
        ],
        compiler_params=pltpu.CompilerParams(needs_layout_passes=False),
    )
    agg = pl.kernel(
        _agg_body,
        mesh=mesh,
        out_type=jax.ShapeDtypeStruct((NC, NH, D), jnp.float32),
        scratch_types=[
            pltpu.VMEM((CPT_MAX, CH), jnp.int32),
            pltpu.VMEM((CPT_MAX, CH), jnp.int32),
            pltpu.VMEM((CH, D), jnp.float32),
            pltpu.VMEM_SHARED((NH, D), jnp.float32),
            pltpu.SemaphoreType.DMA,
        ],
    )
    return deg, agg


BLK = 1000
GRID = N // BLK


def _row_spec(blk, width):
    return pl.BlockSpec((blk, width), lambda i: (i, 0))


def _full_spec(shape):
    return pl.BlockSpec(shape, lambda i: tuple(0 for _ in shape))


def _h0_body(x_ref, wi_ref, bi_ref, out_ref):
    h = jnp.dot(x_ref[...], wi_ref[...], preferred_element_type=jnp.float32)
    out_ref[...] = jnp.maximum(h + bi_ref[...], 0.0)


def _apply_bn(h, hn_ref, stats_ref, gamma_ref, beta_ref):
    mean = stats_ref[0:1, :] * (1.0 / N)
    var = stats_ref[1:2, :] * (1.0 / N) - mean * mean
    scale = gamma_ref[...] * lax.rsqrt(var + 1e-5)
    bias = beta_ref[...] - mean * scale
    return h + hn_ref[...] * scale + bias


def _msg_body(apply_bn, *refs):
    if apply_bn:
        (h_ref, hn_ref, stats_ref, gamma_ref, beta_ref,
         cen_ref, sig_ref, wflat_ref, br_ref, deg_ref, ho_ref, msg_ref) = refs
        h = _apply_bn(h_ref[...], hn_ref, stats_ref, gamma_ref, beta_ref)
    else:
        (h_ref, cen_ref, sig_ref, wflat_ref, br_ref, deg_ref,
         ho_ref, msg_ref) = refs
        h = h_ref[...]
    ho_ref[...] = h
    cen = cen_ref[...]
    sig = sig_ref[...]
    cols = []
    for r in range(R):
        diff = (h - cen[r:r + 1, :]) / sig[r:r + 1, :]
        mu = jnp.exp(-0.5 * diff * diff)
        cols.append(jnp.mean(mu, axis=1, keepdims=True))
    firing = jnp.concatenate(cols, axis=1)
    fn = firing / (jnp.sum(firing, axis=1, keepdims=True) + 1e-12)
    z = jnp.concatenate([fn[:, r:r + 1] * h for r in range(R)], axis=1)
    msg = jnp.dot(z, wflat_ref[...], preferred_element_type=jnp.float32)
    msg = msg + jnp.dot(fn, br_ref[...], preferred_element_type=jnp.float32)
    msg_ref[...] = msg * lax.rsqrt(deg_ref[...] + 1.0)


def _stats_body(p0_ref, p1_ref, deg_ref, hn_ref, stats_ref):
    a = (p0_ref[...] + p1_ref[...]) * lax.rsqrt(deg_ref[...] + 1.0)
    hn = jnp.maximum(a, 0.0)
    hn_ref[...] = hn
    s1 = jnp.sum(hn, axis=0, keepdims=True)
    s2 = jnp.sum(hn * hn, axis=0, keepdims=True)
    block = jnp.concatenate([s1, s2, jnp.zeros((6, D), jnp.float32)], axis=0)

    @pl.when(pl.program_id(0) == 0)
    def _():
        stats_ref[...] = jnp.zeros_like(stats_ref)

    stats_ref[...] += block


def _final_body(h_ref, hn_ref, stats_ref, gamma_ref, beta_ref,
                wh_ref, bh_ref, out_ref):
    h = _apply_bn(h_ref[...], hn_ref, stats_ref, gamma_ref, beta_ref)
    logits = jnp.dot(h, wh_ref[...], preferred_element_type=jnp.float32)
    logits = logits + bh_ref[...]
    m = jnp.max(logits, axis=1, keepdims=True)
    e = jnp.exp(logits - m)
    out_ref[...] = e / jnp.sum(e, axis=1, keepdims=True)


def _h0_call(x, Wi, bi):
    return pl.pallas_call(
        _h0_body,
        grid=(GRID,),
        in_specs=[_row_spec(BLK, D), _full_spec((D, D)), _full_spec((1, D))],
        out_specs=_row_spec(BLK, D),
        out_shape=jax.ShapeDtypeStruct((N, D), jnp.float32),
    )(x, Wi, bi)


def _msg_call(apply_bn, args):
    if apply_bn:
        in_specs = [_row_spec(BLK, D), _row_spec(BLK, D), _full_spec((8, D)),
                    _full_spec((1, D)), _full_spec((1, D))]
    else:
        in_specs = [_row_spec(BLK, D)]
    in_specs += [_full_spec((R, D)), _full_spec((R, D)),
                 _full_spec((R * D, D)), _full_spec((R, D)),
                 _row_spec(BLK, 1)]
    return pl.pallas_call(
        functools.partial(_msg_body, apply_bn),
        grid=(GRID,),
        in_specs=in_specs,
        out_specs=[_row_spec(BLK, D), _row_spec(BLK, D)],
        out_shape=[jax.ShapeDtypeStruct((N, D), jnp.float32),
                   jax.ShapeDtypeStruct((N, D), jnp.float32)],
    )(*args)


def _stats_call(p0, p1, deg_dst):
    return pl.pallas_call(
        _stats_body,
        grid=(GRID,),
        in_specs=[_row_spec(BLK, D), _row_spec(BLK, D), _row_spec(BLK, 1)],
        out_specs=[_row_spec(BLK, D), _full_spec((8, D))],
        out_shape=[jax.ShapeDtypeStruct((N, D), jnp.float32),
                   jax.ShapeDtypeStruct((8, D), jnp.float32)],
    )(p0, p1, deg_dst)


def _final_call(h, hn, stats, gamma, beta, Wh, bh):
    return pl.pallas_call(
        _final_body,
        grid=(GRID,),
        in_specs=[_row_spec(BLK, D), _row_spec(BLK, D), _full_spec((8, D)),
                  _full_spec((1, D)), _full_spec((1, D)),
                  _full_spec((D, DOUT)), _full_spec((1, DOUT))],
        out_specs=_row_spec(BLK, DOUT),
        out_shape=jax.ShapeDtypeStruct((N, DOUT), jnp.float32),
    )(h, hn, stats, gamma, beta, Wh, bh)


def kernel(x, edge_index, Wi, bi, centers, sigmas, Wr, br, gamma, beta, Wh, bh):
    src = edge_index[0]
    dst = edge_index[1]
    npad = EP - E
    pad_discard = jnp.full((npad,), PAD_ROW, jnp.int32)
    pad_zero = jnp.zeros((npad,), jnp.int32)
    src_agg = jnp.concatenate([src, pad_zero]).reshape(NROWS, CH)
    src_p = jnp.concatenate([src, pad_discard])
    dst_p = jnp.concatenate([dst, pad_discard])
    dst_sc = dst_p.reshape(NROWS, CH)
    src_tiles = src_p.reshape(NC * NS, EPT)
    dst_tiles = dst_p.reshape(NC * NS, EPT)

    zeros_agg = jnp.zeros((NH // NS, D), jnp.float32)

    deg_kernel, agg_kernel = _sc_kernels()
    hist = deg_kernel(src_tiles, dst_tiles).sum(axis=0)
    deg_src = hist[:N].reshape(N, 1)
    deg_dst = hist[NHIST:NHIST + N].reshape(N, 1)

    bi2 = bi.reshape(1, D)
    bh2 = bh.reshape(1, DOUT)

    h = _h0_call(x, Wi, bi2)
    hn = None
    stats = None
    for l in range(3):
        cen = centers[l]
        sig = sigmas[l]
        wflat = Wr[l].reshape(R * D, D)
        brl = br[l]
        if l == 0:
            h, msg = _msg_call(False, (h, cen, sig, wflat, brl, deg_src))
        else:
            h, msg = _msg_call(True, (h, hn, stats,
                                      gamma[l - 1].reshape(1, D),
                                      beta[l - 1].reshape(1, D),
                                      cen, sig, wflat, brl, deg_src))
        part = agg_kernel(msg, src_agg, dst_sc, zeros_agg)
        hn, stats = _stats_call(part[0, :N], part[1, :N], deg_dst)
    return _final_call(h, hn, stats, gamma[2].reshape(1, D),
                       beta[2].reshape(1, D), Wh, bh2)
